# Initial kernel scaffold; baseline (speedup 1.0000x reference)
import jax
import jax.numpy as jnp
from jax.experimental import pallas as pl


def kernel(x, last_update, edge_index, t, msg, time_w, time_b, q_w1, q_b1, k_w1, k_b1, v_w1, v_b1, e_w1, s_w1, s_b1, q_w2, q_b2, k_w2, k_b2, v_w2, v_b2, e_w2, s_w2, s_b2):
    raise NotImplementedError("write your pallas kernel here")


# speedup vs baseline: 4.4808x; 4.4808x over previous
import functools
import math

import jax
import jax.numpy as jnp
from jax import lax
from jax.experimental import pallas as pl
from jax.experimental.pallas import tpu as pltpu
from jax.experimental.pallas import tpu_sc as plsc

N = 10000
E = 320000
D = 128
MSG = 16
TD = 32
H = 4
C = 32
HC = H * C
EDGE_DIM = MSG + TD

_SC_INFO = plsc.get_sparse_core_info()
NC = _SC_INFO.num_cores
NS = _SC_INFO.num_subcores
NW = NC * NS
CH = 128
NCHUNKS = E // CH
CPW = -(-NCHUNKS // NW)
NPAD = 10240
ROWS_PER_TILE = NPAD // NS

_mesh = plsc.VectorSubcoreMesh(core_axis_name="c", subcore_axis_name="s")

BN = 2000
BE = 2000


def _proj_body_l1(x_ref, lu_ref, wq_ref, bq_ref, wk_ref, bk_ref, wv_ref,
                  bv_ref, ws_ref, bs_ref, xq_o, xkv_o, xs_o):
    x = x_ref[...]
    xq_o[...] = jnp.dot(x, wq_ref[...], preferred_element_type=jnp.float32) + bq_ref[...]
    xkv_o[:, :HC] = jnp.dot(x, wk_ref[...], preferred_element_type=jnp.float32) + bk_ref[...]
    xkv_o[:, HC:2 * HC] = jnp.dot(x, wv_ref[...], preferred_element_type=jnp.float32) + bv_ref[...]
    xkv_o[:, 2 * HC:] = jnp.broadcast_to(lu_ref[...], (x.shape[0], HC))
    xs_o[...] = jnp.dot(x, ws_ref[...], preferred_element_type=jnp.float32) + bs_ref[...]


def _proj_body_l2(x_ref, wq_ref, bq_ref, wk_ref, bk_ref, wv_ref, bv_ref,
                  ws_ref, bs_ref, xq_o, xkv_o, xs_o):
    x = x_ref[...]
    xq_o[...] = jnp.dot(x, wq_ref[...], preferred_element_type=jnp.float32) + bq_ref[...]
    xkv_o[:, :HC] = jnp.dot(x, wk_ref[...], preferred_element_type=jnp.float32) + bk_ref[...]
    xkv_o[:, HC:] = jnp.dot(x, wv_ref[...], preferred_element_type=jnp.float32) + bv_ref[...]
    xs_o[...] = jnp.dot(x, ws_ref[...], preferred_element_type=jnp.float32) + bs_ref[...]


def _head_onehot():
    lane_head = jax.lax.broadcasted_iota(jnp.int32, (HC, H), 0) // C
    head = jax.lax.broadcasted_iota(jnp.int32, (HC, H), 1)
    return (lane_head == head).astype(jnp.float32)


def _head_onehot_t():
    lane_head = jax.lax.broadcasted_iota(jnp.int32, (H, HC), 1) // C
    head = jax.lax.broadcasted_iota(jnp.int32, (H, HC), 0)
    return (lane_head == head).astype(jnp.float32)


def _edge_math(q, k, v, rel, msg, tw, tb, we):
    enc = jnp.cos(rel * tw + tb)
    attr = jnp.concatenate([enc, msg], axis=1)
    e = jnp.dot(attr, we, preferred_element_type=jnp.float32)
    k = k + e
    v = v + e
    alpha = jnp.dot(q * k, _head_onehot(),
                    preferred_element_type=jnp.float32) * (1.0 / math.sqrt(C))
    w = jnp.exp(alpha)
    wb = jnp.dot(w, _head_onehot_t(), preferred_element_type=jnp.float32)
    return v * wb, wb


def _edge_body_l1(q_ref, kvlu_ref, t_ref, msg_ref, tw_ref, tb_ref, we_ref,
                  wv_o, wb_o, rel_o):
    rel = kvlu_ref[:, 2 * HC:2 * HC + 1] - t_ref[...]
    wv, wb = _edge_math(q_ref[...], kvlu_ref[:, :HC], kvlu_ref[:, HC:2 * HC],
                        rel, msg_ref[...], tw_ref[...], tb_ref[...], we_ref[...])
    wv_o[...] = wv
    wb_o[...] = wb
    rel_o[...] = rel


def _edge_body_l2(q_ref, kv_ref, rel_ref, msg_ref, tw_ref, tb_ref, we_ref,
                  wv_o, wb_o):
    wv, wb = _edge_math(q_ref[...], kv_ref[:, :HC], kv_ref[:, HC:],
                        rel_ref[...], msg_ref[...], tw_ref[...], tb_ref[...],
                        we_ref[...])
    wv_o[...] = wv
    wb_o[...] = wb


def _combine_body(nump_ref, denp_ref, xs_ref, out_ref):
    num = nump_ref[0] + nump_ref[1]
    den = denp_ref[0] + denp_ref[1]
    out_ref[...] = num / (den + 1e-16) + xs_ref[...]


def _full(shape):
    return pl.BlockSpec(shape, lambda i: (0,) * len(shape))


def _tc_proj_l1(x, lu, wq, bq, wk, bk, wv, bv, ws, bs):
    return pl.pallas_call(
        _proj_body_l1,
        grid=(N // BN,),
        in_specs=[pl.BlockSpec((BN, D), lambda i: (i, 0)),
                  pl.BlockSpec((BN, 1), lambda i: (i, 0)),
                  _full((D, HC)), _full((1, HC)), _full((D, HC)), _full((1, HC)),
                  _full((D, HC)), _full((1, HC)), _full((D, HC)), _full((1, HC))],
        out_specs=[pl.BlockSpec((BN, HC), lambda i: (i, 0)),
                   pl.BlockSpec((BN, 3 * HC), lambda i: (i, 0)),
                   pl.BlockSpec((BN, HC), lambda i: (i, 0))],
        out_shape=[jax.ShapeDtypeStruct((N, HC), jnp.float32),
                   jax.ShapeDtypeStruct((N, 3 * HC), jnp.float32),
                   jax.ShapeDtypeStruct((N, HC), jnp.float32)],
    )(x, lu.reshape(N, 1), wq, bq.reshape(1, HC), wk, bk.reshape(1, HC),
      wv, bv.reshape(1, HC), ws, bs.reshape(1, HC))


def _tc_proj_l2(x, wq, bq, wk, bk, wv, bv, ws, bs):
    return pl.pallas_call(
        _proj_body_l2,
        grid=(N // BN,),
        in_specs=[pl.BlockSpec((BN, D), lambda i: (i, 0)),
                  _full((D, HC)), _full((1, HC)), _full((D, HC)), _full((1, HC)),
                  _full((D, HC)), _full((1, HC)), _full((D, HC)), _full((1, HC))],
        out_specs=[pl.BlockSpec((BN, HC), lambda i: (i, 0)),
                   pl.BlockSpec((BN, 2 * HC), lambda i: (i, 0)),
                   pl.BlockSpec((BN, HC), lambda i: (i, 0))],
        out_shape=[jax.ShapeDtypeStruct((N, HC), jnp.float32),
                   jax.ShapeDtypeStruct((N, 2 * HC), jnp.float32),
                   jax.ShapeDtypeStruct((N, HC), jnp.float32)],
    )(x, wq, bq.reshape(1, HC), wk, bk.reshape(1, HC), wv, bv.reshape(1, HC),
      ws, bs.reshape(1, HC))


def _tc_edge_l1(q_rows, kvlu_rows, t, msg, time_w, time_b, e_w):
    return pl.pallas_call(
        _edge_body_l1,
        grid=(E // BE,),
        in_specs=[pl.BlockSpec((BE, HC), lambda i: (i, 0)),
                  pl.BlockSpec((BE, 3 * HC), lambda i: (i, 0)),
                  pl.BlockSpec((BE, 1), lambda i: (i, 0)),
                  pl.BlockSpec((BE, MSG), lambda i: (i, 0)),
                  _full((1, TD)), _full((1, TD)), _full((EDGE_DIM, HC))],
        out_specs=[pl.BlockSpec((BE, HC), lambda i: (i, 0)),
                   pl.BlockSpec((BE, HC), lambda i: (i, 0)),
                   pl.BlockSpec((BE, 1), lambda i: (i, 0))],
        out_shape=[jax.ShapeDtypeStruct((E, HC), jnp.float32),
                   jax.ShapeDtypeStruct((E, HC), jnp.float32),
                   jax.ShapeDtypeStruct((E, 1), jnp.float32)],
    )(q_rows, kvlu_rows, t.reshape(E, 1), msg,
      time_w.reshape(1, TD), time_b.reshape(1, TD), e_w)


def _tc_edge_l2(q_rows, kv_rows, rel, msg, time_w, time_b, e_w):
    return pl.pallas_call(
        _edge_body_l2,
        grid=(E // BE,),
        in_specs=[pl.BlockSpec((BE, HC), lambda i: (i, 0)),
                  pl.BlockSpec((BE, 2 * HC), lambda i: (i, 0)),
                  pl.BlockSpec((BE, 1), lambda i: (i, 0)),
                  pl.BlockSpec((BE, MSG), lambda i: (i, 0)),
                  _full((1, TD)), _full((1, TD)), _full((EDGE_DIM, HC))],
        out_specs=[pl.BlockSpec((BE, HC), lambda i: (i, 0)),
                   pl.BlockSpec((BE, HC), lambda i: (i, 0))],
        out_shape=[jax.ShapeDtypeStruct((E, HC), jnp.float32),
                   jax.ShapeDtypeStruct((E, HC), jnp.float32)],
    )(q_rows, kv_rows, rel, msg,
      time_w.reshape(1, TD), time_b.reshape(1, TD), e_w)


def _tc_combine(numer_p, den_p, xs):
    return pl.pallas_call(
        _combine_body,
        grid=(N // BN,),
        in_specs=[pl.BlockSpec((NC, BN, HC), lambda i: (0, i, 0)),
                  pl.BlockSpec((NC, BN, HC), lambda i: (0, i, 0)),
                  pl.BlockSpec((BN, HC), lambda i: (i, 0))],
        out_specs=pl.BlockSpec((BN, HC), lambda i: (i, 0)),
        out_shape=jax.ShapeDtypeStruct((N, HC), jnp.float32),
    )(numer_p, den_p, xs)


def _gather_body(xq_hbm, xkv_hbm, dst_hbm, src_hbm,
                 qo_hbm, kvo_hbm,
                 dsti, srci, qbuf, kvbuf, sem1, sem2):
    wid = lax.axis_index("s") * NC + lax.axis_index("c")

    def body(i, carry):
        g = wid + i * NW

        @pl.when(g < NCHUNKS)
        def _():
            base = g * CH
            pltpu.sync_copy(dst_hbm.at[pl.ds(base, CH)], dsti)
            pltpu.sync_copy(src_hbm.at[pl.ds(base, CH)], srci)
            cp1 = pltpu.async_copy(xq_hbm.at[dsti], qbuf, sem1)
            cp2 = pltpu.async_copy(xkv_hbm.at[srci], kvbuf, sem2)
            cp1.wait()
            cp2.wait()
            pltpu.sync_copy(qbuf, qo_hbm.at[pl.ds(base, CH)])
            pltpu.sync_copy(kvbuf, kvo_hbm.at[pl.ds(base, CH)])

        return carry

    lax.fori_loop(0, CPW, body, 0)


def _make_sc_gather(kv_width):
    return functools.partial(
        pl.kernel,
        _gather_body,
        out_type=[jax.ShapeDtypeStruct((E, HC), jnp.float32),
                  jax.ShapeDtypeStruct((E, kv_width), jnp.float32)],
        mesh=_mesh,
        scratch_types=[pltpu.VMEM((CH,), jnp.int32),
                       pltpu.VMEM((CH,), jnp.int32),
                       pltpu.VMEM((CH, HC), jnp.float32),
                       pltpu.VMEM((CH, kv_width), jnp.float32),
                       pltpu.SemaphoreType.DMA,
                       pltpu.SemaphoreType.DMA],
    )()


_sc_gather_l1 = _make_sc_gather(3 * HC)
_sc_gather_l2 = _make_sc_gather(2 * HC)


def _scatter_body(rows_hbm, dst_hbm, zeros_hbm, out_hbm, idxb, rowb, acc):
    c = lax.axis_index("c")
    s = lax.axis_index("s")
    wid = s * NC + c
    r0 = s * ROWS_PER_TILE
    pltpu.sync_copy(zeros_hbm.at[pl.ds(r0, ROWS_PER_TILE)],
                    acc.at[pl.ds(r0, ROWS_PER_TILE)])
    plsc.subcore_barrier()

    def body(i, carry):
        g = wid + i * NW

        @pl.when(g < NCHUNKS)
        def _():
            base = g * CH
            pltpu.sync_copy(dst_hbm.at[pl.ds(base, CH)], idxb.at[0])
            pltpu.sync_copy(rows_hbm.at[pl.ds(base, CH)], rowb)
            pltpu.sync_copy(rowb, acc.at[idxb.at[0]], add=True)

        return carry

    lax.fori_loop(0, CPW, body, 0)
    plsc.subcore_barrier()
    pltpu.sync_copy(acc.at[pl.ds(r0, ROWS_PER_TILE)],
                    out_hbm.at[c].at[pl.ds(r0, ROWS_PER_TILE)])


_sc_scatter = functools.partial(
    pl.kernel,
    _scatter_body,
    out_type=jax.ShapeDtypeStruct((NC, NPAD, HC), jnp.float32),
    mesh=_mesh,
    scratch_types=[pltpu.VMEM((1, CH), jnp.int32),
                   pltpu.VMEM((CH, HC), jnp.float32),
                   pltpu.VMEM_SHARED((NPAD, HC), jnp.float32)],
)()


def kernel(x, last_update, edge_index, t, msg, time_w, time_b,
           q_w1, q_b1, k_w1, k_b1, v_w1, v_b1, e_w1, s_w1, s_b1,
           q_w2, q_b2, k_w2, k_b2, v_w2, v_b2, e_w2, s_w2, s_b2):
    src = edge_index[0].astype(jnp.int32)
    dst = edge_index[1].astype(jnp.int32)
    zeros = jnp.zeros((NPAD, HC), jnp.float32)

    xq, xkvlu, xs = _tc_proj_l1(x, last_update, q_w1, q_b1, k_w1, k_b1,
                                v_w1, v_b1, s_w1, s_b1)
    q_rows, kvlu_rows = _sc_gather_l1(xq, xkvlu, dst, src)
    wv, wb, rel = _tc_edge_l1(q_rows, kvlu_rows, t, msg, time_w, time_b, e_w1)
    numer_p = _sc_scatter(wv, dst, zeros)
    den_p = _sc_scatter(wb, dst, zeros)
    h = _tc_combine(numer_p, den_p, xs)

    xq, xkv, xs = _tc_proj_l2(h, q_w2, q_b2, k_w2, k_b2, v_w2, v_b2, s_w2, s_b2)
    q_rows, kv_rows = _sc_gather_l2(xq, xkv, dst, src)
    wv, wb = _tc_edge_l2(q_rows, kv_rows, rel, msg, time_w, time_b, e_w2)
    numer_p = _sc_scatter(wv, dst, zeros)
    den_p = _sc_scatter(wb, dst, zeros)
    return _tc_combine(numer_p, den_p, xs)


# speedup vs baseline: 4.5325x; 1.0116x over previous
import functools
import math

import jax
import jax.numpy as jnp
from jax import lax
from jax.experimental import pallas as pl
from jax.experimental.pallas import tpu as pltpu
from jax.experimental.pallas import tpu_sc as plsc

N = 10000
E = 320000
D = 128
MSG = 16
TD = 32
H = 4
C = 32
HC = H * C
EDGE_DIM = MSG + TD

_SC_INFO = plsc.get_sparse_core_info()
NC = _SC_INFO.num_cores
NS = _SC_INFO.num_subcores
NW = NC * NS
CH = 128
NCHUNKS = E // CH
CPW = -(-NCHUNKS // NW)
NPAD = 10240
ROWS_PER_TILE = NPAD // NS

_mesh = plsc.VectorSubcoreMesh(core_axis_name="c", subcore_axis_name="s")

BN = 2000
BE = 2000


def _proj_body_l1(x_ref, lu_ref, wq_ref, bq_ref, wk_ref, bk_ref, wv_ref,
                  bv_ref, ws_ref, bs_ref, xq_o, xkv_o, xs_o):
    x = x_ref[...]
    xq_o[...] = jnp.dot(x, wq_ref[...], preferred_element_type=jnp.float32) + bq_ref[...]
    xkv_o[:, :HC] = jnp.dot(x, wk_ref[...], preferred_element_type=jnp.float32) + bk_ref[...]
    xkv_o[:, HC:2 * HC] = jnp.dot(x, wv_ref[...], preferred_element_type=jnp.float32) + bv_ref[...]
    xkv_o[:, 2 * HC:] = jnp.broadcast_to(lu_ref[...], (x.shape[0], HC))
    xs_o[...] = jnp.dot(x, ws_ref[...], preferred_element_type=jnp.float32) + bs_ref[...]


def _proj_body_l2(x_ref, wq_ref, bq_ref, wk_ref, bk_ref, wv_ref, bv_ref,
                  ws_ref, bs_ref, xq_o, xkv_o, xs_o):
    x = x_ref[...]
    xq_o[...] = jnp.dot(x, wq_ref[...], preferred_element_type=jnp.float32) + bq_ref[...]
    xkv_o[:, :HC] = jnp.dot(x, wk_ref[...], preferred_element_type=jnp.float32) + bk_ref[...]
    xkv_o[:, HC:] = jnp.dot(x, wv_ref[...], preferred_element_type=jnp.float32) + bv_ref[...]
    xs_o[...] = jnp.dot(x, ws_ref[...], preferred_element_type=jnp.float32) + bs_ref[...]


def _head_onehot():
    lane_head = jax.lax.broadcasted_iota(jnp.int32, (HC, H), 0) // C
    head = jax.lax.broadcasted_iota(jnp.int32, (HC, H), 1)
    return (lane_head == head).astype(jnp.float32)


def _head_onehot_t():
    lane_head = jax.lax.broadcasted_iota(jnp.int32, (H, HC), 1) // C
    head = jax.lax.broadcasted_iota(jnp.int32, (H, HC), 0)
    return (lane_head == head).astype(jnp.float32)


def _edge_math(q, k, v, rel, msg, tw, tb, we):
    enc = jnp.cos(rel * tw + tb)
    attr = jnp.concatenate([enc, msg], axis=1)
    e = jnp.dot(attr, we, preferred_element_type=jnp.float32)
    k = k + e
    v = v + e
    alpha = jnp.dot(q * k, _head_onehot(),
                    preferred_element_type=jnp.float32) * (1.0 / math.sqrt(C))
    w = jnp.exp(alpha)
    wb = jnp.dot(w, _head_onehot_t(), preferred_element_type=jnp.float32)
    return v * wb, wb


def _edge_body_l1(q_ref, kvlu_ref, t_ref, msg_ref, tw_ref, tb_ref, we_ref,
                  wv_o, wb_o, rel_o):
    rel = kvlu_ref[:, 2 * HC:2 * HC + 1] - t_ref[...]
    wv, wb = _edge_math(q_ref[...], kvlu_ref[:, :HC], kvlu_ref[:, HC:2 * HC],
                        rel, msg_ref[...], tw_ref[...], tb_ref[...], we_ref[...])
    wv_o[...] = wv
    wb_o[...] = wb
    rel_o[...] = rel


def _edge_body_l2(q_ref, kv_ref, rel_ref, msg_ref, tw_ref, tb_ref, we_ref,
                  wv_o, wb_o):
    wv, wb = _edge_math(q_ref[...], kv_ref[:, :HC], kv_ref[:, HC:],
                        rel_ref[...], msg_ref[...], tw_ref[...], tb_ref[...],
                        we_ref[...])
    wv_o[...] = wv
    wb_o[...] = wb


def _combine_body(acc_ref, xs_ref, out_ref):
    num = acc_ref[0]
    den = acc_ref[1]
    out_ref[...] = num / (den + 1e-16) + xs_ref[...]


def _full(shape):
    return pl.BlockSpec(shape, lambda i: (0,) * len(shape))


def _tc_proj_l1(x, lu, wq, bq, wk, bk, wv, bv, ws, bs):
    return pl.pallas_call(
        _proj_body_l1,
        grid=(N // BN,),
        in_specs=[pl.BlockSpec((BN, D), lambda i: (i, 0)),
                  pl.BlockSpec((BN, 1), lambda i: (i, 0)),
                  _full((D, HC)), _full((1, HC)), _full((D, HC)), _full((1, HC)),
                  _full((D, HC)), _full((1, HC)), _full((D, HC)), _full((1, HC))],
        out_specs=[pl.BlockSpec((BN, HC), lambda i: (i, 0)),
                   pl.BlockSpec((BN, 3 * HC), lambda i: (i, 0)),
                   pl.BlockSpec((BN, HC), lambda i: (i, 0))],
        out_shape=[jax.ShapeDtypeStruct((N, HC), jnp.float32),
                   jax.ShapeDtypeStruct((N, 3 * HC), jnp.float32),
                   jax.ShapeDtypeStruct((N, HC), jnp.float32)],
    )(x, lu.reshape(N, 1), wq, bq.reshape(1, HC), wk, bk.reshape(1, HC),
      wv, bv.reshape(1, HC), ws, bs.reshape(1, HC))


def _tc_proj_l2(x, wq, bq, wk, bk, wv, bv, ws, bs):
    return pl.pallas_call(
        _proj_body_l2,
        grid=(N // BN,),
        in_specs=[pl.BlockSpec((BN, D), lambda i: (i, 0)),
                  _full((D, HC)), _full((1, HC)), _full((D, HC)), _full((1, HC)),
                  _full((D, HC)), _full((1, HC)), _full((D, HC)), _full((1, HC))],
        out_specs=[pl.BlockSpec((BN, HC), lambda i: (i, 0)),
                   pl.BlockSpec((BN, 2 * HC), lambda i: (i, 0)),
                   pl.BlockSpec((BN, HC), lambda i: (i, 0))],
        out_shape=[jax.ShapeDtypeStruct((N, HC), jnp.float32),
                   jax.ShapeDtypeStruct((N, 2 * HC), jnp.float32),
                   jax.ShapeDtypeStruct((N, HC), jnp.float32)],
    )(x, wq, bq.reshape(1, HC), wk, bk.reshape(1, HC), wv, bv.reshape(1, HC),
      ws, bs.reshape(1, HC))


def _tc_edge_l1(q_rows, kvlu_rows, t, msg, time_w, time_b, e_w):
    return pl.pallas_call(
        _edge_body_l1,
        grid=(E // BE,),
        in_specs=[pl.BlockSpec((BE, HC), lambda i: (i, 0)),
                  pl.BlockSpec((BE, 3 * HC), lambda i: (i, 0)),
                  pl.BlockSpec((BE, 1), lambda i: (i, 0)),
                  pl.BlockSpec((BE, MSG), lambda i: (i, 0)),
                  _full((1, TD)), _full((1, TD)), _full((EDGE_DIM, HC))],
        out_specs=[pl.BlockSpec((BE, HC), lambda i: (i, 0)),
                   pl.BlockSpec((BE, HC), lambda i: (i, 0)),
                   pl.BlockSpec((BE, 1), lambda i: (i, 0))],
        out_shape=[jax.ShapeDtypeStruct((E, HC), jnp.float32),
                   jax.ShapeDtypeStruct((E, HC), jnp.float32),
                   jax.ShapeDtypeStruct((E, 1), jnp.float32)],
    )(q_rows, kvlu_rows, t.reshape(E, 1), msg,
      time_w.reshape(1, TD), time_b.reshape(1, TD), e_w)


def _tc_edge_l2(q_rows, kv_rows, rel, msg, time_w, time_b, e_w):
    return pl.pallas_call(
        _edge_body_l2,
        grid=(E // BE,),
        in_specs=[pl.BlockSpec((BE, HC), lambda i: (i, 0)),
                  pl.BlockSpec((BE, 2 * HC), lambda i: (i, 0)),
                  pl.BlockSpec((BE, 1), lambda i: (i, 0)),
                  pl.BlockSpec((BE, MSG), lambda i: (i, 0)),
                  _full((1, TD)), _full((1, TD)), _full((EDGE_DIM, HC))],
        out_specs=[pl.BlockSpec((BE, HC), lambda i: (i, 0)),
                   pl.BlockSpec((BE, HC), lambda i: (i, 0))],
        out_shape=[jax.ShapeDtypeStruct((E, HC), jnp.float32),
                   jax.ShapeDtypeStruct((E, HC), jnp.float32)],
    )(q_rows, kv_rows, rel, msg,
      time_w.reshape(1, TD), time_b.reshape(1, TD), e_w)


def _tc_combine(acc, xs):
    return pl.pallas_call(
        _combine_body,
        grid=(N // BN,),
        in_specs=[pl.BlockSpec((NC, BN, HC), lambda i: (0, i, 0)),
                  pl.BlockSpec((BN, HC), lambda i: (i, 0))],
        out_specs=pl.BlockSpec((BN, HC), lambda i: (i, 0)),
        out_shape=jax.ShapeDtypeStruct((N, HC), jnp.float32),
    )(acc, xs)


def _gather_body(xq_hbm, xkv_hbm, dst_hbm, src_hbm,
                 qo_hbm, kvo_hbm,
                 dsti, srci, qbuf, kvbuf, sem1, sem2):
    wid = lax.axis_index("s") * NC + lax.axis_index("c")

    def body(i, carry):
        g = wid + i * NW

        @pl.when(g < NCHUNKS)
        def _():
            base = g * CH
            pltpu.sync_copy(dst_hbm.at[pl.ds(base, CH)], dsti)
            pltpu.sync_copy(src_hbm.at[pl.ds(base, CH)], srci)
            cp1 = pltpu.async_copy(xq_hbm.at[dsti], qbuf, sem1)
            cp2 = pltpu.async_copy(xkv_hbm.at[srci], kvbuf, sem2)
            cp1.wait()
            cp2.wait()
            pltpu.sync_copy(qbuf, qo_hbm.at[pl.ds(base, CH)])
            pltpu.sync_copy(kvbuf, kvo_hbm.at[pl.ds(base, CH)])

        return carry

    lax.fori_loop(0, CPW, body, 0)


def _make_sc_gather(kv_width):
    return functools.partial(
        pl.kernel,
        _gather_body,
        out_type=[jax.ShapeDtypeStruct((E, HC), jnp.float32),
                  jax.ShapeDtypeStruct((E, kv_width), jnp.float32)],
        mesh=_mesh,
        scratch_types=[pltpu.VMEM((CH,), jnp.int32),
                       pltpu.VMEM((CH,), jnp.int32),
                       pltpu.VMEM((CH, HC), jnp.float32),
                       pltpu.VMEM((CH, kv_width), jnp.float32),
                       pltpu.SemaphoreType.DMA,
                       pltpu.SemaphoreType.DMA],
    )()


_sc_gather_l1 = _make_sc_gather(3 * HC)
_sc_gather_l2 = _make_sc_gather(2 * HC)


CPT = -(-NCHUNKS // NS)


def _scatter_body(wv_hbm, wb_hbm, dst_hbm, zeros_hbm, out_hbm, idxb, rowb, acc):
    c = lax.axis_index("c")
    s = lax.axis_index("s")
    r0 = s * ROWS_PER_TILE
    pltpu.sync_copy(zeros_hbm.at[pl.ds(r0, ROWS_PER_TILE)],
                    acc.at[pl.ds(r0, ROWS_PER_TILE)])
    plsc.subcore_barrier()

    def run(rows_hbm):
        def body(i, carry):
            g = s * CPT + i

            @pl.when(g < NCHUNKS)
            def _():
                base = g * CH
                pltpu.sync_copy(dst_hbm.at[pl.ds(base, CH)], idxb.at[0])
                pltpu.sync_copy(rows_hbm.at[pl.ds(base, CH)], rowb)
                pltpu.sync_copy(rowb, acc.at[idxb.at[0]], add=True)

            return carry

        lax.fori_loop(0, CPT, body, 0)

    @pl.when(c == 0)
    def _():
        run(wv_hbm)

    @pl.when(c == 1)
    def _():
        run(wb_hbm)

    plsc.subcore_barrier()
    pltpu.sync_copy(acc.at[pl.ds(r0, ROWS_PER_TILE)],
                    out_hbm.at[c].at[pl.ds(r0, ROWS_PER_TILE)])


_sc_scatter = functools.partial(
    pl.kernel,
    _scatter_body,
    out_type=jax.ShapeDtypeStruct((NC, NPAD, HC), jnp.float32),
    mesh=_mesh,
    scratch_types=[pltpu.VMEM((1, CH), jnp.int32),
                   pltpu.VMEM((CH, HC), jnp.float32),
                   pltpu.VMEM_SHARED((NPAD, HC), jnp.float32)],
)()


def kernel(x, last_update, edge_index, t, msg, time_w, time_b,
           q_w1, q_b1, k_w1, k_b1, v_w1, v_b1, e_w1, s_w1, s_b1,
           q_w2, q_b2, k_w2, k_b2, v_w2, v_b2, e_w2, s_w2, s_b2):
    src = edge_index[0].astype(jnp.int32)
    dst = edge_index[1].astype(jnp.int32)
    zeros = jnp.zeros((NPAD, HC), jnp.float32)

    xq, xkvlu, xs = _tc_proj_l1(x, last_update, q_w1, q_b1, k_w1, k_b1,
                                v_w1, v_b1, s_w1, s_b1)
    q_rows, kvlu_rows = _sc_gather_l1(xq, xkvlu, dst, src)
    wv, wb, rel = _tc_edge_l1(q_rows, kvlu_rows, t, msg, time_w, time_b, e_w1)
    acc = _sc_scatter(wv, wb, dst, zeros)
    h = _tc_combine(acc, xs)

    xq, xkv, xs = _tc_proj_l2(h, q_w2, q_b2, k_w2, k_b2, v_w2, v_b2, s_w2, s_b2)
    q_rows, kv_rows = _sc_gather_l2(xq, xkv, dst, src)
    wv, wb = _tc_edge_l2(q_rows, kv_rows, rel, msg, time_w, time_b, e_w2)
    acc = _sc_scatter(wv, wb, dst, zeros)
    return _tc_combine(acc, xs)
